# Initial kernel scaffold; baseline (speedup 1.0000x reference)
import jax
import jax.numpy as jnp
from jax.experimental import pallas as pl


def kernel(x_nchw, w1_taps, b1, w2_taps, b2, s1, s2, fc1_mat, fc1_b, fc2_mat, fc2_b):
    raise NotImplementedError("write your pallas kernel here")


# speedup vs baseline: 4.6229x; 4.6229x over previous
import numpy as np
import jax
import jax.numpy as jnp
from jax.experimental import pallas as pl
from jax.experimental.pallas import tpu as pltpu

_B = 256

_F32 = jnp.float32


def _np_idx_conv1():
    y, dy, dx, c, wo, par = np.meshgrid(
        np.arange(2), np.arange(3), np.arange(3), np.arange(16),
        np.arange(14), np.arange(2), indexing="ij")
    ox = 2 * wo + par
    rows = (y + dy) * 30 + ox + dx
    cols = y * 448 + par * 224 + c * 14 + wo
    taps = dy * 3 + dx
    return rows.ravel(), cols.ravel(), taps.ravel(), c.ravel()


_C1_ROWS, _C1_COLS, _C1_TAPS, _C1_CH = _np_idx_conv1()


def _np_idx_conv2():
    y, dy, dx, cin, c2, wo7, par = np.meshgrid(
        np.arange(2), np.arange(3), np.arange(3), np.arange(16),
        np.arange(32), np.arange(7), np.arange(2), indexing="ij")
    ox2 = 2 * wo7 + par
    wi = ox2 + dx - 1
    valid = (wi >= 0) & (wi < 14)
    rows = ((y + dy) * 224 + cin * 14 + wi)[valid]
    cols = (y * 448 + par * 224 + c2 * 7 + wo7)[valid]
    taps = (dy * 3 + dx)[valid]
    return rows, cols, taps, cin[valid], c2[valid]


_C2_ROWS, _C2_COLS, _C2_TAPS, _C2_CIN, _C2_CO = _np_idx_conv2()

_H, _W, _C = np.meshgrid(np.arange(7), np.arange(7), np.arange(32),
                         indexing="ij")
_FC1_NEW = (_H * 224 + _C * 7 + _W).ravel()
_FC1_OLD = (_H * 224 + _W * 32 + _C).ravel()


def _prep_mats(w1_taps, b1, w2_taps, b2, fc1_mat):
    w1band = jnp.zeros((120, 896), _F32).at[_C1_ROWS, _C1_COLS].set(
        w1_taps[_C1_TAPS, _C1_CH])
    w2band = jnp.zeros((896, 896), _F32).at[_C2_ROWS, _C2_COLS].set(
        w2_taps[_C2_TAPS, _C2_CIN, _C2_CO])
    fc1p = jnp.zeros((1568, 128), _F32).at[_FC1_NEW].set(fc1_mat[_FC1_OLD])
    b1v = jnp.repeat(b1.reshape(16, 1), 14, axis=1).reshape(1, 224)
    b2v = jnp.repeat(b2.reshape(32, 1), 7, axis=1).reshape(1, 224)
    return w1band, w2band, fc1p, b1v, b2v


def _fused_kernel(x_ref, w1b_ref, b1_ref, w2b_ref, b2_ref,
                  fc1_ref, fc1b_ref, fc2_ref, fc2b_ref, o_ref):
    X = x_ref[...]
    w1b = w1b_ref[...]
    b1v = b1_ref[...]
    w2b = w2b_ref[...]
    b2v = b2_ref[...]

    B = X.shape[0]
    zrow = jnp.zeros((B, 224), _F32)

    p1rows = [zrow]
    for h in range(14):
        u = jnp.dot(X[:, 60 * h:60 * h + 120], w1b,
                    preferred_element_type=_F32)
        v = jnp.maximum(u[:, :448], u[:, 448:])
        vh = jnp.maximum(v[:, :224], v[:, 224:])
        p1rows.append(jnp.maximum(vh + b1v, 0.0))
    p1rows.append(zrow)
    P1 = jnp.concatenate(p1rows, axis=1)

    p2rows = []
    for h in range(7):
        u = jnp.dot(P1[:, 448 * h:448 * h + 896], w2b,
                    preferred_element_type=_F32)
        v = jnp.maximum(u[:, :448], u[:, 448:])
        vh = jnp.maximum(v[:, :224], v[:, 224:])
        p2rows.append(jnp.maximum(vh + b2v, 0.0))
    P2 = jnp.concatenate(p2rows, axis=1)

    hmid = jnp.dot(P2, fc1_ref[...], preferred_element_type=_F32)
    hmid = jnp.maximum(hmid + fc1b_ref[...], 0.0)
    o_ref[...] = (jnp.dot(hmid, fc2_ref[...], preferred_element_type=_F32)
                  + fc2b_ref[...])


def kernel(x_nchw, w1_taps, b1, w2_taps, b2, s1, s2,
           fc1_mat, fc1_b, fc2_mat, fc2_b):
    del s1, s2
    N = x_nchw.shape[0]
    Np = (N + _B - 1) // _B * _B
    x = x_nchw.astype(_F32)[:, 0, :, :]
    x = jnp.pad(x, ((0, Np - N), (1, 1), (1, 1))).reshape(Np, 900)

    w1band, w2band, fc1p, b1v, b2v = _prep_mats(w1_taps, b1, w2_taps, b2,
                                                fc1_mat)

    out = pl.pallas_call(
        _fused_kernel,
        out_shape=jax.ShapeDtypeStruct((Np, 10), _F32),
        grid=(Np // _B,),
        in_specs=[
            pl.BlockSpec((_B, 900), lambda i: (i, 0)),
            pl.BlockSpec((120, 896), lambda i: (0, 0)),
            pl.BlockSpec((1, 224), lambda i: (0, 0)),
            pl.BlockSpec((896, 896), lambda i: (0, 0)),
            pl.BlockSpec((1, 224), lambda i: (0, 0)),
            pl.BlockSpec((1568, 128), lambda i: (0, 0)),
            pl.BlockSpec((1, 128), lambda i: (0, 0)),
            pl.BlockSpec((128, 10), lambda i: (0, 0)),
            pl.BlockSpec((1, 10), lambda i: (0, 0)),
        ],
        out_specs=pl.BlockSpec((_B, 10), lambda i: (i, 0)),
        compiler_params=pltpu.CompilerParams(
            dimension_semantics=("parallel",),
            vmem_limit_bytes=64 * 1024 * 1024),
    )(x, w1band, b1v, w2band, b2v, fc1p, fc1_b, fc2_mat, fc2_b)
    return out[:N]


# speedup vs baseline: 28.6896x; 6.2059x over previous
import numpy as np
import jax
import jax.numpy as jnp
from jax.experimental import pallas as pl
from jax.experimental.pallas import tpu as pltpu

_B = 256

_F32 = jnp.float32

_PY = (np.arange(4)[None, :, None] - np.arange(2)[None, None, :]
       == np.arange(3)[:, None, None]).astype(np.float32)
_Q1 = (np.arange(32)[None, :, None, None]
       - 2 * np.arange(14)[None, None, None, :]
       - np.arange(2)[None, None, :, None]
       == np.arange(3)[:, None, None, None]).astype(np.float32)
_Q2 = (np.arange(14)[None, :, None, None]
       - 2 * np.arange(7)[None, None, None, :]
       - np.arange(2)[None, None, :, None] + 1
       == np.arange(3)[:, None, None, None]).astype(np.float32)


def _prep_mats(w1_taps, b1, w2_taps, b2, fc1_mat):
    w1t = w1_taps.reshape(3, 3, 16)
    w2t = w2_taps.reshape(3, 3, 16, 32)
    w1band = jnp.einsum("aIY,bXPW,abc->IXYPcW", _PY, _Q1, w1t)
    w1band = w1band.reshape(128, 2, 2, 224)
    w1band = jnp.pad(w1band, ((0, 0), (0, 0), (0, 0), (0, 32)))
    w1band = w1band.reshape(128, 1024)
    w2band = jnp.einsum("aRY,bXPW,abio->RiXYPoW", _PY, _Q2, w2t)
    w2band = w2band.reshape(4, 224, 2, 2, 224)
    w2band = jnp.pad(w2band, ((0, 0), (0, 32), (0, 0), (0, 0), (0, 32)))
    w2band = w2band.reshape(1024, 1024)
    fc1p = fc1_mat.reshape(7, 7, 32, 128).transpose(0, 2, 1, 3)
    fc1p = jnp.pad(fc1p.reshape(7, 224, 128), ((0, 0), (0, 32), (0, 0)))
    fc1p = fc1p.reshape(1792, 128)
    b1v = jnp.pad(jnp.repeat(b1.reshape(16, 1), 14, axis=1).reshape(1, 224),
                  ((0, 0), (0, 32)))
    b2v = jnp.pad(jnp.repeat(b2.reshape(32, 1), 7, axis=1).reshape(1, 224),
                  ((0, 0), (0, 32)))
    return w1band, w2band, fc1p, b1v, b2v


def _fused_kernel(x_ref, w1b_ref, b1_ref, w2b_ref, b2_ref,
                  fc1_ref, fc1b_ref, fc2_ref, fc2b_ref, o_ref):
    X = x_ref[...]
    w1b = w1b_ref[...]
    b1v = b1_ref[...]
    w2b = w2b_ref[...]
    b2v = b2_ref[...]

    B = X.shape[0]
    zrow = jnp.zeros((B, 256), _F32)

    p1rows = [zrow]
    for h in range(14):
        u = jnp.dot(X[:, 64 * h:64 * h + 128], w1b,
                    preferred_element_type=_F32)
        v = jnp.maximum(u[:, :512], u[:, 512:])
        vh = jnp.maximum(v[:, :256], v[:, 256:])
        p1rows.append(jnp.maximum(vh + b1v, 0.0))
    p1rows.append(zrow)
    P1 = jnp.concatenate(p1rows, axis=1)

    p2rows = []
    for h in range(7):
        u = jnp.dot(P1[:, 512 * h:512 * h + 1024], w2b,
                    preferred_element_type=_F32)
        v = jnp.maximum(u[:, :512], u[:, 512:])
        vh = jnp.maximum(v[:, :256], v[:, 256:])
        p2rows.append(jnp.maximum(vh + b2v, 0.0))
    P2 = jnp.concatenate(p2rows, axis=1)

    hmid = jnp.dot(P2, fc1_ref[...], preferred_element_type=_F32)
    hmid = jnp.maximum(hmid + fc1b_ref[...], 0.0)
    o_ref[...] = (jnp.dot(hmid, fc2_ref[...], preferred_element_type=_F32)
                  + fc2b_ref[...])


def kernel(x_nchw, w1_taps, b1, w2_taps, b2, s1, s2,
           fc1_mat, fc1_b, fc2_mat, fc2_b):
    del s1, s2
    N = x_nchw.shape[0]
    Np = (N + _B - 1) // _B * _B
    x = x_nchw.astype(_F32)[:, 0, :, :]
    x = jnp.pad(x, ((0, Np - N), (1, 1), (1, 3))).reshape(Np, 960)

    w1band, w2band, fc1p, b1v, b2v = _prep_mats(w1_taps, b1, w2_taps, b2,
                                                fc1_mat)

    out = pl.pallas_call(
        _fused_kernel,
        out_shape=jax.ShapeDtypeStruct((Np, 10), _F32),
        grid=(Np // _B,),
        in_specs=[
            pl.BlockSpec((_B, 960), lambda i: (i, 0)),
            pl.BlockSpec((128, 1024), lambda i: (0, 0)),
            pl.BlockSpec((1, 256), lambda i: (0, 0)),
            pl.BlockSpec((1024, 1024), lambda i: (0, 0)),
            pl.BlockSpec((1, 256), lambda i: (0, 0)),
            pl.BlockSpec((1792, 128), lambda i: (0, 0)),
            pl.BlockSpec((1, 128), lambda i: (0, 0)),
            pl.BlockSpec((128, 10), lambda i: (0, 0)),
            pl.BlockSpec((1, 10), lambda i: (0, 0)),
        ],
        out_specs=pl.BlockSpec((_B, 10), lambda i: (i, 0)),
        compiler_params=pltpu.CompilerParams(
            dimension_semantics=("parallel",),
            vmem_limit_bytes=64 * 1024 * 1024),
    )(x, w1band, b1v, w2band, b2v, fc1p, fc1_b, fc2_mat, fc2_b)
    return out[:N]


# speedup vs baseline: 36.0718x; 1.2573x over previous
import numpy as np
import jax
import jax.numpy as jnp
from jax.experimental import pallas as pl
from jax.experimental.pallas import tpu as pltpu

_B = 256

_F32 = jnp.float32

_PY = (np.arange(4)[None, :, None] - np.arange(2)[None, None, :]
       == np.arange(3)[:, None, None]).astype(np.float32)
_Q1 = (np.arange(32)[None, :, None, None]
       - 2 * np.arange(14)[None, None, None, :]
       - np.arange(2)[None, None, :, None]
       == np.arange(3)[:, None, None, None]).astype(np.float32)
_Q2 = (np.arange(14)[None, :, None, None]
       - 2 * np.arange(7)[None, None, None, :]
       - np.arange(2)[None, None, :, None] + 1
       == np.arange(3)[:, None, None, None]).astype(np.float32)


def _prep_mats(w1_taps, b1, w2_taps, b2, fc1_mat):
    w1t = w1_taps.reshape(3, 3, 16)
    w2t = w2_taps.reshape(3, 3, 16, 32)
    w1band = jnp.einsum("aIY,bXPW,abc->IXYPcW", _PY, _Q1, w1t)
    w1band = w1band.reshape(128, 2, 2, 224)
    w1band = jnp.pad(w1band, ((0, 0), (0, 0), (0, 0), (0, 32)))
    w1band = w1band.reshape(128, 1024)
    w2band = jnp.einsum("bXPW,abio->aiXPoW", _Q2, w2t)
    w2band = w2band.reshape(3, 224, 2, 224)
    w2band = jnp.pad(w2band, ((0, 0), (0, 32), (0, 0), (0, 32)))
    w2band = w2band.reshape(768, 512)
    fc1p = fc1_mat.reshape(7, 7, 32, 128).transpose(0, 2, 1, 3)
    fc1p = jnp.pad(fc1p.reshape(7, 224, 128), ((0, 0), (0, 32), (0, 0)))
    fc1p = fc1p.reshape(1792, 128)
    b1v = jnp.pad(jnp.repeat(b1.reshape(16, 1), 14, axis=1).reshape(1, 224),
                  ((0, 0), (0, 32)))
    b2v = jnp.pad(jnp.repeat(b2.reshape(32, 1), 7, axis=1).reshape(1, 224),
                  ((0, 0), (0, 32)))
    return w1band, w2band, fc1p, b1v, b2v


def _fused_kernel(x_ref, w1b_ref, b1_ref, w2b_ref, b2_ref,
                  fc1_ref, fc1b_ref, fc2_ref, fc2b_ref, o_ref):
    X = x_ref[...]
    w1b = w1b_ref[...]
    b1v = b1_ref[...]
    w2b = w2b_ref[...]
    b2v = b2_ref[...]

    B = X.shape[0]
    zrow = jnp.zeros((B, 256), _F32)

    p1rows = [zrow]
    for h in range(14):
        u = jnp.dot(X[:, 64 * h:64 * h + 128], w1b,
                    preferred_element_type=_F32)
        v = jnp.maximum(u[:, :512], u[:, 512:])
        vh = jnp.maximum(v[:, :256], v[:, 256:])
        p1rows.append(jnp.maximum(vh + b1v, 0.0))
    p1rows.append(zrow)
    P1 = jnp.concatenate(p1rows, axis=1)

    urows = [jnp.dot(P1[:, 256 * h:256 * h + 768], w2b,
                     preferred_element_type=_F32)
             for h in range(14)]
    p2rows = []
    for h in range(7):
        v = jnp.maximum(urows[2 * h], urows[2 * h + 1])
        vh = jnp.maximum(v[:, :256], v[:, 256:])
        p2rows.append(jnp.maximum(vh + b2v, 0.0))
    P2 = jnp.concatenate(p2rows, axis=1)

    hmid = jnp.dot(P2, fc1_ref[...], preferred_element_type=_F32)
    hmid = jnp.maximum(hmid + fc1b_ref[...], 0.0)
    o_ref[...] = (jnp.dot(hmid, fc2_ref[...], preferred_element_type=_F32)
                  + fc2b_ref[...])


def kernel(x_nchw, w1_taps, b1, w2_taps, b2, s1, s2,
           fc1_mat, fc1_b, fc2_mat, fc2_b):
    del s1, s2
    N = x_nchw.shape[0]
    Np = (N + _B - 1) // _B * _B
    x = x_nchw.astype(_F32)[:, 0, :, :]
    x = jnp.pad(x, ((0, Np - N), (1, 1), (1, 3))).reshape(Np, 960)

    w1band, w2band, fc1p, b1v, b2v = _prep_mats(w1_taps, b1, w2_taps, b2,
                                                fc1_mat)

    out = pl.pallas_call(
        _fused_kernel,
        out_shape=jax.ShapeDtypeStruct((Np, 10), _F32),
        grid=(Np // _B,),
        in_specs=[
            pl.BlockSpec((_B, 960), lambda i: (i, 0)),
            pl.BlockSpec((128, 1024), lambda i: (0, 0)),
            pl.BlockSpec((1, 256), lambda i: (0, 0)),
            pl.BlockSpec((768, 512), lambda i: (0, 0)),
            pl.BlockSpec((1, 256), lambda i: (0, 0)),
            pl.BlockSpec((1792, 128), lambda i: (0, 0)),
            pl.BlockSpec((1, 128), lambda i: (0, 0)),
            pl.BlockSpec((128, 10), lambda i: (0, 0)),
            pl.BlockSpec((1, 10), lambda i: (0, 0)),
        ],
        out_specs=pl.BlockSpec((_B, 10), lambda i: (i, 0)),
        compiler_params=pltpu.CompilerParams(
            dimension_semantics=("parallel",),
            vmem_limit_bytes=64 * 1024 * 1024),
    )(x, w1band, b1v, w2band, b2v, fc1p, fc1_b, fc2_mat, fc2_b)
    return out[:N]


# speedup vs baseline: 36.9409x; 1.0241x over previous
import numpy as np
import jax
import jax.numpy as jnp
from jax.experimental import pallas as pl
from jax.experimental.pallas import tpu as pltpu

_B = 512

_F32 = jnp.float32

_PY = (np.arange(4)[None, :, None] - np.arange(2)[None, None, :]
       == np.arange(3)[:, None, None]).astype(np.float32)
_Q1 = (np.arange(32)[None, :, None, None]
       - 2 * np.arange(14)[None, None, None, :]
       - np.arange(2)[None, None, :, None]
       == np.arange(3)[:, None, None, None]).astype(np.float32)
_Q2 = (np.arange(14)[None, :, None, None]
       - 2 * np.arange(7)[None, None, None, :]
       - np.arange(2)[None, None, :, None] + 1
       == np.arange(3)[:, None, None, None]).astype(np.float32)


def _prep_mats(w1_taps, b1, w2_taps, b2, fc1_mat):
    w1t = w1_taps.reshape(3, 3, 16)
    w2t = w2_taps.reshape(3, 3, 16, 32)
    w1band = jnp.einsum("aIY,bXPW,abc->IXYPcW", _PY, _Q1, w1t)
    w1band = w1band.reshape(128, 2, 2, 224)
    w1band = jnp.pad(w1band, ((0, 0), (0, 0), (0, 0), (0, 32)))
    w1band = w1band.reshape(128, 1024)
    w2band = jnp.einsum("bXPW,abio->aiXPoW", _Q2, w2t)
    w2band = w2band.reshape(3, 224, 2, 224)
    w2band = jnp.pad(w2band, ((0, 0), (0, 32), (0, 0), (0, 32)))
    w2band = w2band.reshape(768, 512)
    fc1p = fc1_mat.reshape(7, 7, 32, 128).transpose(0, 2, 1, 3)
    fc1p = jnp.pad(fc1p.reshape(7, 224, 128), ((0, 0), (0, 32), (0, 0)))
    fc1p = fc1p.reshape(1792, 128)
    b1v = jnp.pad(jnp.repeat(b1.reshape(16, 1), 14, axis=1).reshape(1, 224),
                  ((0, 0), (0, 32)))
    b2v = jnp.pad(jnp.repeat(b2.reshape(32, 1), 7, axis=1).reshape(1, 224),
                  ((0, 0), (0, 32)))
    return w1band, w2band, fc1p, b1v, b2v


def _fused_kernel(x_ref, w1b_ref, b1_ref, w2b_ref, b2_ref,
                  fc1_ref, fc1b_ref, fc2_ref, fc2b_ref, o_ref):
    X = x_ref[...]
    w1b = w1b_ref[...]
    b1v = b1_ref[...]
    w2b = w2b_ref[...]
    b2v = b2_ref[...]

    B = X.shape[0]
    zrow = jnp.zeros((B, 256), _F32)

    p1rows = [zrow]
    for h in range(14):
        u = jnp.dot(X[:, 64 * h:64 * h + 128], w1b,
                    preferred_element_type=_F32)
        v = jnp.maximum(u[:, :512], u[:, 512:])
        vh = jnp.maximum(v[:, :256], v[:, 256:])
        p1rows.append(jnp.maximum(vh + b1v, 0.0))
    p1rows.append(zrow)
    P1 = jnp.concatenate(p1rows, axis=1)

    urows = [jnp.dot(P1[:, 256 * h:256 * h + 768], w2b,
                     preferred_element_type=_F32)
             for h in range(14)]
    p2rows = []
    for h in range(7):
        v = jnp.maximum(urows[2 * h], urows[2 * h + 1])
        vh = jnp.maximum(v[:, :256], v[:, 256:])
        p2rows.append(jnp.maximum(vh + b2v, 0.0))
    P2 = jnp.concatenate(p2rows, axis=1)

    hmid = jnp.dot(P2, fc1_ref[...], preferred_element_type=_F32)
    hmid = jnp.maximum(hmid + fc1b_ref[...], 0.0)
    o_ref[...] = (jnp.dot(hmid, fc2_ref[...], preferred_element_type=_F32)
                  + fc2b_ref[...])


def kernel(x_nchw, w1_taps, b1, w2_taps, b2, s1, s2,
           fc1_mat, fc1_b, fc2_mat, fc2_b):
    del s1, s2
    N = x_nchw.shape[0]
    Np = (N + _B - 1) // _B * _B
    x = x_nchw.astype(_F32)[:, 0, :, :]
    x = jnp.pad(x, ((0, Np - N), (1, 1), (1, 3))).reshape(Np, 960)

    w1band, w2band, fc1p, b1v, b2v = _prep_mats(w1_taps, b1, w2_taps, b2,
                                                fc1_mat)

    out = pl.pallas_call(
        _fused_kernel,
        out_shape=jax.ShapeDtypeStruct((Np, 10), _F32),
        grid=(Np // _B,),
        in_specs=[
            pl.BlockSpec((_B, 960), lambda i: (i, 0)),
            pl.BlockSpec((128, 1024), lambda i: (0, 0)),
            pl.BlockSpec((1, 256), lambda i: (0, 0)),
            pl.BlockSpec((768, 512), lambda i: (0, 0)),
            pl.BlockSpec((1, 256), lambda i: (0, 0)),
            pl.BlockSpec((1792, 128), lambda i: (0, 0)),
            pl.BlockSpec((1, 128), lambda i: (0, 0)),
            pl.BlockSpec((128, 10), lambda i: (0, 0)),
            pl.BlockSpec((1, 10), lambda i: (0, 0)),
        ],
        out_specs=pl.BlockSpec((_B, 10), lambda i: (i, 0)),
        compiler_params=pltpu.CompilerParams(
            dimension_semantics=("parallel",),
            vmem_limit_bytes=64 * 1024 * 1024),
    )(x, w1band, b1v, w2band, b2v, fc1p, fc1_b, fc2_mat, fc2_b)
    return out[:N]
